# baseline (speedup 1.0000x reference)
import functools

import jax
import jax.numpy as jnp
from jax import lax
from jax.experimental import pallas as pl
from jax.experimental.pallas import tpu as pltpu
from jax.experimental.pallas import tpu_sc as plsc

N_TOKENS = 16384
NUM_CODES = 8192
DIM = 64

N_BLK = 512
K_BLK = 1024
N_GRID = N_TOKENS // N_BLK
K_GRID = NUM_CODES // K_BLK
GROUP = 4096
GROUP_BLKS = GROUP // K_BLK


def _argmin_body(a2_ref, z_ref, e_ref, b2_ref, idx_ref, best_val, best_idx):
    j = pl.program_id(1)
    z = z_ref[...].astype(jnp.bfloat16)
    e = e_ref[...].astype(jnp.bfloat16)
    ab = lax.dot_general(z, e, (((1,), (1,)), ((), ())),
                         preferred_element_type=jnp.float32)
    dist = (a2_ref[...] - 2.0 * ab) + b2_ref[...]
    m = jnp.min(dist, axis=1, keepdims=True)
    cols = lax.broadcasted_iota(jnp.int32, dist.shape, 1)
    amin = jnp.min(jnp.where(dist == m, cols, K_BLK), axis=1,
                   keepdims=True) + j * K_BLK

    @pl.when(j == 0)
    def _init():
        best_val[...] = m
        best_idx[...] = amin

    @pl.when(j > 0)
    def _update():
        better = m < best_val[...]
        best_val[...] = jnp.where(better, m, best_val[...])
        best_idx[...] = jnp.where(better, amin, best_idx[...])

    @pl.when(jnp.logical_and((j + 1) % GROUP_BLKS == 0, j != K_GRID - 1))
    def _round_group():
        best_val[...] = best_val[...].astype(jnp.bfloat16).astype(jnp.float32)

    @pl.when(j == K_GRID - 1)
    def _emit():
        idx_ref[...] = best_idx[...]


_argmin_call = pl.pallas_call(
    _argmin_body,
    grid=(N_GRID, K_GRID),
    in_specs=[
        pl.BlockSpec((N_BLK, 1), lambda i, j: (i, 0)),
        pl.BlockSpec((N_BLK, DIM), lambda i, j: (i, 0)),
        pl.BlockSpec((K_BLK, DIM), lambda i, j: (j, 0)),
        pl.BlockSpec((1, K_BLK), lambda i, j: (0, j)),
    ],
    out_specs=pl.BlockSpec((N_BLK, 1), lambda i, j: (i, 0)),
    out_shape=jax.ShapeDtypeStruct((N_TOKENS, 1), jnp.int32),
    scratch_shapes=[
        pltpu.VMEM((N_BLK, 1), jnp.float32),
        pltpu.VMEM((N_BLK, 1), jnp.int32),
    ],
    compiler_params=pltpu.CompilerParams(
        dimension_semantics=("parallel", "arbitrary")),
)


_NC = 2
_NS = 16
_NW = _NC * _NS
_B_PER_W = N_TOKENS // _NW


@functools.cache
def _build_gather_rows():
    @functools.partial(
        pl.kernel,
        mesh=plsc.VectorSubcoreMesh(core_axis_name="c", subcore_axis_name="s"),
        out_type=jax.ShapeDtypeStruct((N_TOKENS, DIM), jnp.float32),
        scratch_types=[
            pltpu.VMEM((_B_PER_W,), jnp.int32),
            pltpu.VMEM((_B_PER_W, DIM), jnp.float32),
            pltpu.SemaphoreType.DMA,
        ],
        compiler_params=pltpu.CompilerParams(use_tc_tiling_on_sc=False),
    )
    def _gather_rows(table_hbm, idx_hbm, out_hbm, idx_v, rows_v, sem):
        wid = lax.axis_index("s") * _NC + lax.axis_index("c")
        base = wid * _B_PER_W
        pltpu.sync_copy(idx_hbm.at[pl.ds(base, _B_PER_W)], idx_v)
        pltpu.async_copy(table_hbm.at[idx_v], rows_v, sem).wait()
        pltpu.sync_copy(rows_v, out_hbm.at[pl.ds(base, _B_PER_W)])

    return _gather_rows


def kernel(z_flat, embedding):
    a2 = jnp.sum(z_flat ** 2, axis=1, keepdims=True)
    b2 = jnp.sum(embedding ** 2, axis=1)
    idx2d = _argmin_call(a2, z_flat, embedding, b2.reshape(1, NUM_CODES))
    indices = idx2d.reshape(N_TOKENS)
    quantized = _build_gather_rows()(embedding, indices)
    return (quantized, indices)


# speedup vs baseline: 1.0460x; 1.0460x over previous
import functools

import jax
import jax.numpy as jnp
from jax import lax
from jax.experimental import pallas as pl
from jax.experimental.pallas import tpu as pltpu
from jax.experimental.pallas import tpu_sc as plsc

N_TOKENS = 16384
NUM_CODES = 8192
DIM = 64

N_BLK = 512
K_BLK = 1024
N_GRID = N_TOKENS // N_BLK
K_GRID = NUM_CODES // K_BLK
GROUP = 4096
GROUP_BLKS = GROUP // K_BLK


LANES = 128
N_CHUNKS = K_BLK // LANES


def _argmin_body(a2_ref, z_ref, e2_ref, b2_ref, idx_ref, best_val, best_idx):
    j = pl.program_id(1)
    z = z_ref[...].astype(jnp.bfloat16)
    e2 = e2_ref[...].astype(jnp.bfloat16)
    s = lax.dot_general(z, e2, (((1,), (1,)), ((), ())),
                        preferred_element_type=jnp.float32)
    dist = (a2_ref[...] + s) + b2_ref[...]
    v = dist[:, 0:LANES]
    c = jnp.zeros((N_BLK, LANES), jnp.int32)
    for t in range(1, N_CHUNKS):
        d_t = dist[:, t * LANES:(t + 1) * LANES]
        lt = d_t < v
        v = jnp.where(lt, d_t, v)
        c = jnp.where(lt, t, c)
    m = jnp.min(v, axis=1, keepdims=True)
    lane = lax.broadcasted_iota(jnp.int32, (N_BLK, LANES), 1)
    idx128 = c * LANES + lane + j * K_BLK
    amin = jnp.min(jnp.where(v == m, idx128, NUM_CODES), axis=1,
                   keepdims=True)

    @pl.when(j == 0)
    def _init():
        best_val[...] = m
        best_idx[...] = amin

    @pl.when(j > 0)
    def _update():
        better = m < best_val[...]
        best_val[...] = jnp.where(better, m, best_val[...])
        best_idx[...] = jnp.where(better, amin, best_idx[...])

    @pl.when(jnp.logical_and((j + 1) % GROUP_BLKS == 0, j != K_GRID - 1))
    def _round_group():
        best_val[...] = best_val[...].astype(jnp.bfloat16).astype(jnp.float32)

    @pl.when(j == K_GRID - 1)
    def _emit():
        idx_ref[...] = best_idx[...]


_argmin_call = pl.pallas_call(
    _argmin_body,
    grid=(N_GRID, K_GRID),
    in_specs=[
        pl.BlockSpec((N_BLK, 1), lambda i, j: (i, 0)),
        pl.BlockSpec((N_BLK, DIM), lambda i, j: (i, 0)),
        pl.BlockSpec((K_BLK, DIM), lambda i, j: (j, 0)),
        pl.BlockSpec((1, K_BLK), lambda i, j: (0, j)),
    ],
    out_specs=pl.BlockSpec((N_BLK, 1), lambda i, j: (i, 0)),
    out_shape=jax.ShapeDtypeStruct((N_TOKENS, 1), jnp.int32),
    scratch_shapes=[
        pltpu.VMEM((N_BLK, 1), jnp.float32),
        pltpu.VMEM((N_BLK, 1), jnp.int32),
    ],
    compiler_params=pltpu.CompilerParams(
        dimension_semantics=("parallel", "arbitrary")),
)


_NC = 2
_NS = 16
_NW = _NC * _NS
_B_PER_W = N_TOKENS // _NW


@functools.cache
def _build_gather_rows():
    @functools.partial(
        pl.kernel,
        mesh=plsc.VectorSubcoreMesh(core_axis_name="c", subcore_axis_name="s"),
        out_type=jax.ShapeDtypeStruct((N_TOKENS, DIM), jnp.float32),
        scratch_types=[
            pltpu.VMEM((_B_PER_W,), jnp.int32),
            pltpu.VMEM((_B_PER_W, DIM), jnp.float32),
            pltpu.SemaphoreType.DMA,
        ],
        compiler_params=pltpu.CompilerParams(use_tc_tiling_on_sc=False),
    )
    def _gather_rows(table_hbm, idx_hbm, out_hbm, idx_v, rows_v, sem):
        wid = lax.axis_index("s") * _NC + lax.axis_index("c")
        base = wid * _B_PER_W
        pltpu.sync_copy(idx_hbm.at[pl.ds(base, _B_PER_W)], idx_v)
        pltpu.async_copy(table_hbm.at[idx_v], rows_v, sem).wait()
        pltpu.sync_copy(rows_v, out_hbm.at[pl.ds(base, _B_PER_W)])

    return _gather_rows


def kernel(z_flat, embedding):
    a2 = jnp.sum(z_flat ** 2, axis=1, keepdims=True)
    b2 = jnp.sum(embedding ** 2, axis=1)
    e2 = embedding * jnp.float32(-2.0)
    idx2d = _argmin_call(a2, z_flat, e2, b2.reshape(1, NUM_CODES))
    indices = idx2d.reshape(N_TOKENS)
    quantized = _build_gather_rows()(embedding, indices)
    return (quantized, indices)
